# baseline (speedup 1.0000x reference)
import functools

import jax
import jax.numpy as jnp
from jax import lax
from jax.experimental import pallas as pl
from jax.experimental.pallas import tpu as pltpu
from jax.experimental.pallas import tpu_sc as plsc

E = 8
DM = 1024
DF = 4096
NT = 2048
K = 2
NR = NT * K

B = 128
CAP = NR + E * B
NB = CAP // B
F = 512
NFF = DF // F
NW = 32
TPW = NT // NW
CW = 32

_SC_MESH = dict(core_axis_name="c", subcore_axis_name="s")


def _routing_meta(expert_indices):
    flat_e = expert_indices.reshape(-1).astype(jnp.int32)
    oh = (flat_e[:, None] == jnp.arange(E, dtype=jnp.int32)[None, :])
    oh = oh.astype(jnp.int32)
    cum = jnp.cumsum(oh, axis=0)
    counts = cum[-1]
    rank = jnp.sum(cum * oh, axis=1) - 1
    padded_counts = ((counts + B - 1) // B) * B
    cum_padded = jnp.cumsum(padded_counts)
    padded_starts = cum_padded - padded_counts
    ppos = jnp.sum(oh * padded_starts[None, :], axis=1) + rank
    block_expert = jnp.searchsorted(
        cum_padded, jnp.arange(NB, dtype=jnp.int32) * B, side="right"
    ).astype(jnp.int32)
    block_expert = jnp.minimum(block_expert, E - 1)
    pw = ppos.reshape(NW, TPW, K).transpose(0, 2, 1)
    return block_expert, pw


def _dispatch_rows(x, pw):
    mesh = plsc.VectorSubcoreMesh(**_SC_MESH)

    @functools.partial(
        pl.kernel,
        mesh=mesh,
        out_type=jax.ShapeDtypeStruct((CAP, DM), jnp.float32),
        scratch_types=[
            pltpu.VMEM((K, TPW), jnp.int32),
            pltpu.VMEM((TPW, DM), jnp.float32),
            pltpu.SemaphoreType.DMA,
        ],
    )
    def dispatch_k(x_hbm, pw_hbm, xs_hbm, idx_v, rows_v, sem):
        wid = lax.axis_index("s") * 2 + lax.axis_index("c")
        pltpu.sync_copy(pw_hbm.at[wid], idx_v)
        pltpu.sync_copy(x_hbm.at[pl.ds(wid * TPW, TPW)], rows_v)
        h1 = pltpu.async_copy(rows_v, xs_hbm.at[idx_v.at[0]], sem)
        h2 = pltpu.async_copy(rows_v, xs_hbm.at[idx_v.at[1]], sem)
        h1.wait()
        h2.wait()

    return dispatch_k(x, pw)


def _ffn_body(be_ref, xs_ref, w1_ref, w2_ref, w3_ref, ys_ref,
              acc_ref, xb16_ref, w12b_ref, w3b_ref):
    j = pl.program_id(0)
    i = pl.program_id(1)
    base = i * B
    prev_be = be_ref[jnp.maximum(i - 1, 0)]
    fresh = jnp.logical_or(i == 0, be_ref[i] != prev_be)

    @pl.when(fresh)
    def _():
        w12b_ref[:, pl.ds(0, F)] = w1_ref[0].astype(jnp.bfloat16)
        w12b_ref[:, pl.ds(F, F)] = w2_ref[0].astype(jnp.bfloat16)
        w3b_ref[...] = w3_ref[0].astype(jnp.bfloat16)

    @pl.when(j == 0)
    def _():
        xb16_ref[pl.ds(base, B), :] = xs_ref[...].astype(jnp.bfloat16)

    xb = xb16_ref[pl.ds(base, B), :]
    gv = jnp.dot(xb, w12b_ref[...], preferred_element_type=jnp.float32)
    g = gv[:, :F]
    v = gv[:, F:]
    h = g * jax.nn.sigmoid(g) * v
    contrib = jnp.dot(h.astype(jnp.bfloat16), w3b_ref[...],
                      preferred_element_type=jnp.float32)

    @pl.when(j == 0)
    def _():
        acc_ref[pl.ds(base, B), :] = contrib

    @pl.when(jnp.logical_and(j != 0, j != NFF - 1))
    def _():
        acc_ref[pl.ds(base, B), :] = acc_ref[pl.ds(base, B), :] + contrib

    @pl.when(j == NFF - 1)
    def _():
        ys_ref[...] = acc_ref[pl.ds(base, B), :] + contrib


def _grouped_ffn(xs, block_expert, w1, w2, w3):
    grid_spec = pltpu.PrefetchScalarGridSpec(
        num_scalar_prefetch=1,
        grid=(NFF, NB),
        in_specs=[
            pl.BlockSpec((B, DM), lambda j, i, be: (jnp.where(j == 0, i, 0), 0)),
            pl.BlockSpec((1, DM, F), lambda j, i, be: (be[i], 0, j)),
            pl.BlockSpec((1, DM, F), lambda j, i, be: (be[i], 0, j)),
            pl.BlockSpec((1, F, DM), lambda j, i, be: (be[i], j, 0)),
        ],
        out_specs=pl.BlockSpec(
            (B, DM), lambda j, i, be: (jnp.where(j == NFF - 1, i, 0), 0)
        ),
        scratch_shapes=[
            pltpu.VMEM((CAP, DM), jnp.float32),
            pltpu.VMEM((CAP, DM), jnp.bfloat16),
            pltpu.VMEM((DM, 2 * F), jnp.bfloat16),
            pltpu.VMEM((F, DM), jnp.bfloat16),
        ],
    )
    return pl.pallas_call(
        _ffn_body,
        grid_spec=grid_spec,
        out_shape=jax.ShapeDtypeStruct((CAP, DM), jnp.float32),
        compiler_params=pltpu.CompilerParams(
            dimension_semantics=("arbitrary", "arbitrary"),
        ),
    )(block_expert, xs, w1, w2, w3)


def _combine(ys, pw, wb):
    mesh = plsc.VectorSubcoreMesh(**_SC_MESH)

    @functools.partial(
        pl.kernel,
        mesh=mesh,
        out_type=jax.ShapeDtypeStruct((NT, DM), jnp.float32),
        scratch_types=[
            pltpu.VMEM((K, TPW), jnp.int32),
            pltpu.VMEM((TPW, 2 * 16), jnp.float32),
            pltpu.VMEM((CW, DM), jnp.float32),
            pltpu.VMEM((CW, DM), jnp.float32),
            pltpu.SemaphoreType.DMA,
        ],
    )
    def combine_k(ys_hbm, pw_hbm, wb_hbm, out_hbm, idx_v, w_v, a_v, b_v, sem):
        wid = lax.axis_index("s") * 2 + lax.axis_index("c")
        base = wid * TPW
        pltpu.sync_copy(pw_hbm.at[wid], idx_v)
        pltpu.sync_copy(wb_hbm.at[pl.ds(base, TPW)], w_v)

        @pl.loop(0, TPW // CW)
        def _(c):
            pltpu.async_copy(
                ys_hbm.at[idx_v.at[0, pl.ds(c * CW, CW)]], a_v, sem
            ).wait()
            pltpu.async_copy(
                ys_hbm.at[idx_v.at[1, pl.ds(c * CW, CW)]], b_v, sem
            ).wait()

            @pl.loop(0, CW)
            def _(r):
                w0 = w_v[c * CW + r, pl.ds(0, 16)]
                w1v = w_v[c * CW + r, pl.ds(16, 16)]

                @pl.loop(0, DM, step=16)
                def _(cc):
                    a_v[r, pl.ds(cc, 16)] = (
                        w0 * a_v[r, pl.ds(cc, 16)]
                        + w1v * b_v[r, pl.ds(cc, 16)]
                    )

            pltpu.sync_copy(a_v, out_hbm.at[pl.ds(base + c * CW, CW)])

    return combine_k(ys, pw, wb)


def kernel(x, expert_indices, expert_weights, w1, w2, w3):
    block_expert, pw = _routing_meta(expert_indices)
    ew = expert_weights.astype(jnp.float32)
    wb = jnp.repeat(ew, 16, axis=1)
    xs = _dispatch_rows(x, pw)
    ys = _grouped_ffn(xs, block_expert, w1, w2, w3)
    return _combine(ys, pw, wb)


# speedup vs baseline: 1.4085x; 1.4085x over previous
import functools

import jax
import jax.numpy as jnp
from jax import lax
from jax.experimental import pallas as pl
from jax.experimental.pallas import tpu as pltpu
from jax.experimental.pallas import tpu_sc as plsc

E = 8
DM = 1024
DF = 4096
NT = 2048
K = 2
NR = NT * K

B = 256
CAP = NR + E * B
NB = CAP // B
F = 1024
NFF = DF // F
NW = 32
TPW = NT // NW
CW = 32

_SC_MESH = dict(core_axis_name="c", subcore_axis_name="s")


def _routing_meta(expert_indices):
    flat_e = expert_indices.reshape(-1).astype(jnp.int32)
    oh = (flat_e[:, None] == jnp.arange(E, dtype=jnp.int32)[None, :])
    oh = oh.astype(jnp.int32)
    cum = jnp.cumsum(oh, axis=0)
    counts = cum[-1]
    rank = jnp.sum(cum * oh, axis=1) - 1
    padded_counts = ((counts + B - 1) // B) * B
    cum_padded = jnp.cumsum(padded_counts)
    padded_starts = cum_padded - padded_counts
    ppos = jnp.sum(oh * padded_starts[None, :], axis=1) + rank
    block_expert = jnp.searchsorted(
        cum_padded, jnp.arange(NB, dtype=jnp.int32) * B, side="right"
    ).astype(jnp.int32)
    block_expert = jnp.minimum(block_expert, E - 1)
    pw = ppos.reshape(NW, TPW, K).transpose(0, 2, 1)
    return block_expert, pw


def _dispatch_rows(x, pw):
    mesh = plsc.VectorSubcoreMesh(**_SC_MESH)

    @functools.partial(
        pl.kernel,
        mesh=mesh,
        out_type=jax.ShapeDtypeStruct((CAP, DM), jnp.float32),
        scratch_types=[
            pltpu.VMEM((K, TPW), jnp.int32),
            pltpu.VMEM((TPW, DM), jnp.float32),
            pltpu.SemaphoreType.DMA,
        ],
    )
    def dispatch_k(x_hbm, pw_hbm, xs_hbm, idx_v, rows_v, sem):
        wid = lax.axis_index("s") * 2 + lax.axis_index("c")
        pltpu.sync_copy(pw_hbm.at[wid], idx_v)
        pltpu.sync_copy(x_hbm.at[pl.ds(wid * TPW, TPW)], rows_v)
        h1 = pltpu.async_copy(rows_v, xs_hbm.at[idx_v.at[0]], sem)
        h2 = pltpu.async_copy(rows_v, xs_hbm.at[idx_v.at[1]], sem)
        h1.wait()
        h2.wait()

    return dispatch_k(x, pw)


def _ffn_body(be_ref, xs_ref, w1_ref, w2_ref, w3_ref, ys_ref,
              acc_ref, w12b_ref, w3b_ref):
    j = pl.program_id(0)
    i = pl.program_id(1)
    base = i * B
    prev_be = be_ref[jnp.maximum(i - 1, 0)]
    fresh = jnp.logical_or(i == 0, be_ref[i] != prev_be)

    @pl.when(fresh)
    def _():
        w12b_ref[:, pl.ds(0, F)] = w1_ref[0].astype(jnp.bfloat16)
        w12b_ref[:, pl.ds(F, F)] = w2_ref[0].astype(jnp.bfloat16)
        w3b_ref[...] = w3_ref[0].astype(jnp.bfloat16)

    xb = xs_ref[...].astype(jnp.bfloat16)
    gv = jnp.dot(xb, w12b_ref[...], preferred_element_type=jnp.float32)
    g = gv[:, :F]
    v = gv[:, F:]
    h = g * jax.nn.sigmoid(g) * v
    contrib = jnp.dot(h.astype(jnp.bfloat16), w3b_ref[...],
                      preferred_element_type=jnp.float32)

    @pl.when(j == 0)
    def _():
        acc_ref[pl.ds(base, B), :] = contrib

    @pl.when(jnp.logical_and(j != 0, j != NFF - 1))
    def _():
        acc_ref[pl.ds(base, B), :] = acc_ref[pl.ds(base, B), :] + contrib

    @pl.when(j == NFF - 1)
    def _():
        ys_ref[...] = acc_ref[pl.ds(base, B), :] + contrib


def _grouped_ffn(xs, block_expert, w1, w2, w3):
    grid_spec = pltpu.PrefetchScalarGridSpec(
        num_scalar_prefetch=1,
        grid=(NFF, NB),
        in_specs=[
            pl.BlockSpec((B, DM), lambda j, i, be: (i, 0)),
            pl.BlockSpec((1, DM, F), lambda j, i, be: (be[i], 0, j)),
            pl.BlockSpec((1, DM, F), lambda j, i, be: (be[i], 0, j)),
            pl.BlockSpec((1, F, DM), lambda j, i, be: (be[i], j, 0)),
        ],
        out_specs=pl.BlockSpec(
            (B, DM), lambda j, i, be: (jnp.where(j == NFF - 1, i, 0), 0)
        ),
        scratch_shapes=[
            pltpu.VMEM((CAP, DM), jnp.float32),
            pltpu.VMEM((DM, 2 * F), jnp.bfloat16),
            pltpu.VMEM((F, DM), jnp.bfloat16),
        ],
    )
    return pl.pallas_call(
        _ffn_body,
        grid_spec=grid_spec,
        out_shape=jax.ShapeDtypeStruct((CAP, DM), jnp.float32),
        compiler_params=pltpu.CompilerParams(
            dimension_semantics=("arbitrary", "arbitrary"),
            vmem_limit_bytes=64 * 1024 * 1024,
        ),
    )(block_expert, xs, w1, w2, w3)


def _combine(ys, pw, wb):
    mesh = plsc.VectorSubcoreMesh(**_SC_MESH)

    @functools.partial(
        pl.kernel,
        mesh=mesh,
        out_type=jax.ShapeDtypeStruct((NT, DM), jnp.float32),
        scratch_types=[
            pltpu.VMEM((K, TPW), jnp.int32),
            pltpu.VMEM((TPW, 2 * 16), jnp.float32),
            pltpu.VMEM((CW, DM), jnp.float32),
            pltpu.VMEM((CW, DM), jnp.float32),
            pltpu.SemaphoreType.DMA,
        ],
    )
    def combine_k(ys_hbm, pw_hbm, wb_hbm, out_hbm, idx_v, w_v, a_v, b_v, sem):
        wid = lax.axis_index("s") * 2 + lax.axis_index("c")
        base = wid * TPW
        pltpu.sync_copy(pw_hbm.at[wid], idx_v)
        pltpu.sync_copy(wb_hbm.at[pl.ds(base, TPW)], w_v)

        @pl.loop(0, TPW // CW)
        def _(c):
            pltpu.async_copy(
                ys_hbm.at[idx_v.at[0, pl.ds(c * CW, CW)]], a_v, sem
            ).wait()
            pltpu.async_copy(
                ys_hbm.at[idx_v.at[1, pl.ds(c * CW, CW)]], b_v, sem
            ).wait()

            @pl.loop(0, CW)
            def _(r):
                w0 = w_v[c * CW + r, pl.ds(0, 16)]
                w1v = w_v[c * CW + r, pl.ds(16, 16)]

                @pl.loop(0, DM, step=16)
                def _(cc):
                    a_v[r, pl.ds(cc, 16)] = (
                        w0 * a_v[r, pl.ds(cc, 16)]
                        + w1v * b_v[r, pl.ds(cc, 16)]
                    )

            pltpu.sync_copy(a_v, out_hbm.at[pl.ds(base + c * CW, CW)])

    return combine_k(ys, pw, wb)


def kernel(x, expert_indices, expert_weights, w1, w2, w3):
    block_expert, pw = _routing_meta(expert_indices)
    ew = expert_weights.astype(jnp.float32)
    wb = jnp.repeat(ew, 16, axis=1)
    xs = _dispatch_rows(x, pw)
    ys = _grouped_ffn(xs, block_expert, w1, w2, w3)
    return _combine(ys, pw, wb)


# speedup vs baseline: 1.4786x; 1.0497x over previous
import functools

import jax
import jax.numpy as jnp
from jax import lax
from jax.experimental import pallas as pl
from jax.experimental.pallas import tpu as pltpu
from jax.experimental.pallas import tpu_sc as plsc

E = 8
DM = 1024
DF = 4096
NT = 2048
K = 2
NR = NT * K

B = 256
CAP = NR + E * B
NB = CAP // B
F = 1024
NFF = DF // F
NW = 32
TPW = NT // NW
CW = 32

_SC_MESH = dict(core_axis_name="c", subcore_axis_name="s")


def _routing_meta(expert_indices):
    flat_e = expert_indices.reshape(-1).astype(jnp.int32)
    oh = (flat_e[:, None] == jnp.arange(E, dtype=jnp.int32)[None, :])
    oh = oh.astype(jnp.int32)
    cum = jnp.cumsum(oh, axis=0)
    counts = cum[-1]
    rank = jnp.sum(cum * oh, axis=1) - 1
    padded_counts = ((counts + B - 1) // B) * B
    cum_padded = jnp.cumsum(padded_counts)
    padded_starts = cum_padded - padded_counts
    ppos = jnp.sum(oh * padded_starts[None, :], axis=1) + rank
    block_expert = jnp.searchsorted(
        cum_padded, jnp.arange(NB, dtype=jnp.int32) * B, side="right"
    ).astype(jnp.int32)
    block_expert = jnp.minimum(block_expert, E - 1)
    pw = ppos.reshape(NW, TPW, K).transpose(0, 2, 1)
    return block_expert, pw


def _dispatch_rows(x, pw):
    mesh = plsc.VectorSubcoreMesh(**_SC_MESH)

    @functools.partial(
        pl.kernel,
        mesh=mesh,
        out_type=jax.ShapeDtypeStruct((CAP, DM), jnp.float32),
        scratch_types=[
            pltpu.VMEM((K, TPW), jnp.int32),
            pltpu.VMEM((TPW, DM), jnp.float32),
            pltpu.SemaphoreType.DMA,
        ],
    )
    def dispatch_k(x_hbm, pw_hbm, xs_hbm, idx_v, rows_v, sem):
        wid = lax.axis_index("s") * 2 + lax.axis_index("c")
        pltpu.sync_copy(pw_hbm.at[wid], idx_v)
        pltpu.sync_copy(x_hbm.at[pl.ds(wid * TPW, TPW)], rows_v)
        h1 = pltpu.async_copy(rows_v, xs_hbm.at[idx_v.at[0]], sem)
        h2 = pltpu.async_copy(rows_v, xs_hbm.at[idx_v.at[1]], sem)
        h1.wait()
        h2.wait()

    return dispatch_k(x, pw)


def _ffn_body(be_ref, xs_ref, w1_ref, w2_ref, w3_ref, ys_ref,
              acc_ref, w12b_ref, w3b_ref):
    j = pl.program_id(0)
    i = pl.program_id(1)
    base = i * B
    prev_be = be_ref[jnp.maximum(i - 1, 0)]
    fresh = jnp.logical_or(i == 0, be_ref[i] != prev_be)

    @pl.when(fresh)
    def _():
        w12b_ref[:, pl.ds(0, F)] = w1_ref[0].astype(jnp.bfloat16)
        w12b_ref[:, pl.ds(F, F)] = w2_ref[0].astype(jnp.bfloat16)
        w3b_ref[...] = w3_ref[0].astype(jnp.bfloat16)

    xb = xs_ref[...].astype(jnp.bfloat16)
    gv = jnp.dot(xb, w12b_ref[...], preferred_element_type=jnp.float32)
    g = gv[:, :F]
    v = gv[:, F:]
    h = g * jax.nn.sigmoid(g) * v
    contrib = jnp.dot(h.astype(jnp.bfloat16), w3b_ref[...],
                      preferred_element_type=jnp.float32)

    @pl.when(j == 0)
    def _():
        acc_ref[pl.ds(base, B), :] = contrib

    @pl.when(jnp.logical_and(j != 0, j != NFF - 1))
    def _():
        acc_ref[pl.ds(base, B), :] = acc_ref[pl.ds(base, B), :] + contrib

    @pl.when(j == NFF - 1)
    def _():
        ys_ref[...] = acc_ref[pl.ds(base, B), :] + contrib


def _grouped_ffn(xs, block_expert, w1, w2, w3):
    grid_spec = pltpu.PrefetchScalarGridSpec(
        num_scalar_prefetch=1,
        grid=(NFF, NB),
        in_specs=[
            pl.BlockSpec((B, DM), lambda j, i, be: (i, 0)),
            pl.BlockSpec((1, DM, F), lambda j, i, be: (be[i], 0, j)),
            pl.BlockSpec((1, DM, F), lambda j, i, be: (be[i], 0, j)),
            pl.BlockSpec((1, F, DM), lambda j, i, be: (be[i], j, 0)),
        ],
        out_specs=pl.BlockSpec(
            (B, DM), lambda j, i, be: (jnp.where(j == NFF - 1, i, 0), 0)
        ),
        scratch_shapes=[
            pltpu.VMEM((CAP, DM), jnp.float32),
            pltpu.VMEM((DM, 2 * F), jnp.bfloat16),
            pltpu.VMEM((F, DM), jnp.bfloat16),
        ],
    )
    return pl.pallas_call(
        _ffn_body,
        grid_spec=grid_spec,
        out_shape=jax.ShapeDtypeStruct((CAP, DM), jnp.float32),
        compiler_params=pltpu.CompilerParams(
            dimension_semantics=("arbitrary", "arbitrary"),
            vmem_limit_bytes=64 * 1024 * 1024,
        ),
    )(block_expert, xs, w1, w2, w3)


def _combine(ys, pw, wb):
    mesh = plsc.VectorSubcoreMesh(**_SC_MESH)

    @functools.partial(
        pl.kernel,
        mesh=mesh,
        out_type=jax.ShapeDtypeStruct((NT, DM), jnp.float32),
        scratch_types=[
            pltpu.VMEM((K, TPW), jnp.int32),
            pltpu.VMEM((TPW, 2 * 16), jnp.float32),
            pltpu.VMEM((CW, DM), jnp.float32),
            pltpu.VMEM((CW, DM), jnp.float32),
            pltpu.SemaphoreType.DMA,
        ],
    )
    def combine_k(ys_hbm, pw_hbm, wb_hbm, out_hbm, idx_v, w_v, a_v, b_v, sem):
        wid = lax.axis_index("s") * 2 + lax.axis_index("c")
        base = wid * TPW
        pltpu.sync_copy(pw_hbm.at[wid], idx_v)
        pltpu.sync_copy(wb_hbm.at[pl.ds(base, TPW)], w_v)

        @pl.loop(0, TPW // CW)
        def _(c):
            ha = pltpu.async_copy(
                ys_hbm.at[idx_v.at[0, pl.ds(c * CW, CW)]], a_v, sem
            )
            hb = pltpu.async_copy(
                ys_hbm.at[idx_v.at[1, pl.ds(c * CW, CW)]], b_v, sem
            )
            ha.wait()
            hb.wait()

            @pl.loop(0, CW)
            def _(r):
                w0 = w_v[c * CW + r, pl.ds(0, 16)]
                w1v = w_v[c * CW + r, pl.ds(16, 16)]

                @pl.loop(0, DM, step=64)
                def _(cc):
                    for u in range(4):
                        sl = pl.ds(cc + u * 16, 16)
                        a_v[r, sl] = w0 * a_v[r, sl] + w1v * b_v[r, sl]

            pltpu.sync_copy(a_v, out_hbm.at[pl.ds(base + c * CW, CW)])

    return combine_k(ys, pw, wb)


def kernel(x, expert_indices, expert_weights, w1, w2, w3):
    block_expert, pw = _routing_meta(expert_indices)
    ew = expert_weights.astype(jnp.float32)
    wb = jnp.repeat(ew, 16, axis=1)
    xs = _dispatch_rows(x, pw)
    ys = _grouped_ffn(xs, block_expert, w1, w2, w3)
    return _combine(ys, pw, wb)


# speedup vs baseline: 1.5185x; 1.0270x over previous
import functools

import jax
import jax.numpy as jnp
from jax import lax
from jax.experimental import pallas as pl
from jax.experimental.pallas import tpu as pltpu
from jax.experimental.pallas import tpu_sc as plsc

E = 8
DM = 1024
DF = 4096
NT = 2048
K = 2
NR = NT * K

B = 256
CAP = NR + E * B
NB = CAP // B
F = 1024
NFF = DF // F
NW = 32
TPW = NT // NW
CW = 32

_SC_MESH = dict(core_axis_name="c", subcore_axis_name="s")


def _routing_meta(expert_indices):
    flat_e = expert_indices.reshape(-1).astype(jnp.int32)
    oh = (flat_e[:, None] == jnp.arange(E, dtype=jnp.int32)[None, :])
    oh = oh.astype(jnp.int32)
    cum = jnp.cumsum(oh, axis=0)
    counts = cum[-1]
    rank = jnp.sum(cum * oh, axis=1) - 1
    padded_counts = ((counts + B - 1) // B) * B
    cum_padded = jnp.cumsum(padded_counts)
    padded_starts = cum_padded - padded_counts
    ppos = jnp.sum(oh * padded_starts[None, :], axis=1) + rank
    block_expert = jnp.searchsorted(
        cum_padded, jnp.arange(NB, dtype=jnp.int32) * B, side="right"
    ).astype(jnp.int32)
    block_expert = jnp.minimum(block_expert, E - 1)
    pw = ppos.reshape(NW, TPW, K).transpose(0, 2, 1)
    return block_expert, pw


def _dispatch_rows(x, pw):
    mesh = plsc.VectorSubcoreMesh(**_SC_MESH)

    @functools.partial(
        pl.kernel,
        mesh=mesh,
        out_type=jax.ShapeDtypeStruct((CAP, DM), jnp.float32),
        scratch_types=[
            pltpu.VMEM((K, TPW), jnp.int32),
            pltpu.VMEM((TPW, DM), jnp.float32),
            pltpu.SemaphoreType.DMA,
        ],
    )
    def dispatch_k(x_hbm, pw_hbm, xs_hbm, idx_v, rows_v, sem):
        wid = lax.axis_index("s") * 2 + lax.axis_index("c")
        pltpu.sync_copy(pw_hbm.at[wid], idx_v)
        pltpu.sync_copy(x_hbm.at[pl.ds(wid * TPW, TPW)], rows_v)
        h1 = pltpu.async_copy(rows_v, xs_hbm.at[idx_v.at[0]], sem)
        h2 = pltpu.async_copy(rows_v, xs_hbm.at[idx_v.at[1]], sem)
        h1.wait()
        h2.wait()

    return dispatch_k(x, pw)


def _ffn_body(be_ref, xs_ref, w1_ref, w2_ref, w3_ref, ys_ref,
              acc_ref, xb16_ref, w12b_ref, w3b_ref):
    j = pl.program_id(0)
    i = pl.program_id(1)
    base = i * B
    prev_be = be_ref[jnp.maximum(i - 1, 0)]
    fresh = jnp.logical_or(i == 0, be_ref[i] != prev_be)

    @pl.when(fresh)
    def _():
        w12b_ref[:, pl.ds(0, F)] = w1_ref[0].astype(jnp.bfloat16)
        w12b_ref[:, pl.ds(F, F)] = w2_ref[0].astype(jnp.bfloat16)
        w3b_ref[...] = w3_ref[0].astype(jnp.bfloat16)

    @pl.when(j == 0)
    def _():
        xb16_ref[pl.ds(base, B), :] = xs_ref[...].astype(jnp.bfloat16)

    xb = xb16_ref[pl.ds(base, B), :]
    gv = jnp.dot(xb, w12b_ref[...], preferred_element_type=jnp.float32)
    g = gv[:, :F]
    v = gv[:, F:]
    h = g * jax.nn.sigmoid(g) * v
    contrib = jnp.dot(h.astype(jnp.bfloat16), w3b_ref[...],
                      preferred_element_type=jnp.float32)

    @pl.when(j == 0)
    def _():
        acc_ref[pl.ds(base, B), :] = contrib.astype(jnp.bfloat16)

    @pl.when(jnp.logical_and(j != 0, j != NFF - 1))
    def _():
        acc_ref[pl.ds(base, B), :] = (
            acc_ref[pl.ds(base, B), :] + contrib.astype(jnp.bfloat16)
        )

    @pl.when(j == NFF - 1)
    def _():
        ys_ref[...] = acc_ref[pl.ds(base, B), :].astype(jnp.float32) + contrib


def _grouped_ffn(xs, block_expert, w1, w2, w3):
    grid_spec = pltpu.PrefetchScalarGridSpec(
        num_scalar_prefetch=1,
        grid=(NFF, NB),
        in_specs=[
            pl.BlockSpec((B, DM), lambda j, i, be: (jnp.where(j == 0, i, 0), 0)),
            pl.BlockSpec((1, DM, F), lambda j, i, be: (be[i], 0, j)),
            pl.BlockSpec((1, DM, F), lambda j, i, be: (be[i], 0, j)),
            pl.BlockSpec((1, F, DM), lambda j, i, be: (be[i], j, 0)),
        ],
        out_specs=pl.BlockSpec(
            (B, DM), lambda j, i, be: (jnp.where(j == NFF - 1, i, 0), 0)
        ),
        scratch_shapes=[
            pltpu.VMEM((CAP, DM), jnp.bfloat16),
            pltpu.VMEM((CAP, DM), jnp.bfloat16),
            pltpu.VMEM((DM, 2 * F), jnp.bfloat16),
            pltpu.VMEM((F, DM), jnp.bfloat16),
        ],
    )
    return pl.pallas_call(
        _ffn_body,
        grid_spec=grid_spec,
        out_shape=jax.ShapeDtypeStruct((CAP, DM), jnp.float32),
        compiler_params=pltpu.CompilerParams(
            dimension_semantics=("arbitrary", "arbitrary"),
            vmem_limit_bytes=64 * 1024 * 1024,
        ),
    )(block_expert, xs, w1, w2, w3)


def _combine(ys, pw, wb):
    mesh = plsc.VectorSubcoreMesh(**_SC_MESH)

    @functools.partial(
        pl.kernel,
        mesh=mesh,
        out_type=jax.ShapeDtypeStruct((NT, DM), jnp.float32),
        scratch_types=[
            pltpu.VMEM((K, TPW), jnp.int32),
            pltpu.VMEM((TPW, 2 * 16), jnp.float32),
            pltpu.VMEM((CW, DM), jnp.float32),
            pltpu.VMEM((CW, DM), jnp.float32),
            pltpu.SemaphoreType.DMA,
        ],
    )
    def combine_k(ys_hbm, pw_hbm, wb_hbm, out_hbm, idx_v, w_v, a_v, b_v, sem):
        wid = lax.axis_index("s") * 2 + lax.axis_index("c")
        base = wid * TPW
        pltpu.sync_copy(pw_hbm.at[wid], idx_v)
        pltpu.sync_copy(wb_hbm.at[pl.ds(base, TPW)], w_v)

        @pl.loop(0, TPW // CW)
        def _(c):
            ha = pltpu.async_copy(
                ys_hbm.at[idx_v.at[0, pl.ds(c * CW, CW)]], a_v, sem
            )
            hb = pltpu.async_copy(
                ys_hbm.at[idx_v.at[1, pl.ds(c * CW, CW)]], b_v, sem
            )
            ha.wait()
            hb.wait()

            @pl.loop(0, CW)
            def _(r):
                w0 = w_v[c * CW + r, pl.ds(0, 16)]
                w1v = w_v[c * CW + r, pl.ds(16, 16)]

                @pl.loop(0, DM, step=64)
                def _(cc):
                    for u in range(4):
                        sl = pl.ds(cc + u * 16, 16)
                        a_v[r, sl] = w0 * a_v[r, sl] + w1v * b_v[r, sl]

            pltpu.sync_copy(a_v, out_hbm.at[pl.ds(base + c * CW, CW)])

    return combine_k(ys, pw, wb)


def kernel(x, expert_indices, expert_weights, w1, w2, w3):
    block_expert, pw = _routing_meta(expert_indices)
    ew = expert_weights.astype(jnp.float32)
    wb = jnp.repeat(ew, 16, axis=1)
    xs = _dispatch_rows(x, pw)
    ys = _grouped_ffn(xs, block_expert, w1, w2, w3)
    return _combine(ys, pw, wb)
